# Initial kernel scaffold; baseline (speedup 1.0000x reference)
import jax
import jax.numpy as jnp
from jax.experimental import pallas as pl


def kernel(x, edge_index, node_ids, label_inds, W1, b1, W2, b2):
    raise NotImplementedError("write your pallas kernel here")


# speedup vs baseline: 18.3831x; 18.3831x over previous
import functools

import jax
import jax.numpy as jnp
from jax import lax
from jax.experimental import pallas as pl
from jax.experimental.pallas import tpu as pltpu
from jax.experimental.pallas import tpu_sc as plsc

N = 10000
E = 320000
D = 128
DL = 16
NID = 2000

NPAD = 10240
NW = 32
CHUNK = 128
NCH = 79
EPT = NCH * CHUNK
EPAD = NW * EPT
RPT = NPAD // 16
NIDPAD = 2048
IDS_PT = NIDPAD // NW

_sc_mesh = plsc.VectorSubcoreMesh(core_axis_name="c", subcore_axis_name="s")
_sc_params = pltpu.CompilerParams(use_tc_tiling_on_sc=False)


def _make_prop(width):
  @functools.partial(
      pl.kernel,
      out_type=jax.ShapeDtypeStruct((2, NPAD, width), jnp.float32),
      mesh=_sc_mesh,
      compiler_params=_sc_params,
      scratch_types=[
          pltpu.VMEM((NCH, CHUNK), jnp.int32),
          pltpu.VMEM((NCH, CHUNK), jnp.int32),
          pltpu.VMEM((CHUNK, width), jnp.float32),
          pltpu.VMEM_SHARED((NPAD, width), jnp.float32),
          pltpu.SemaphoreType.DMA,
      ],
  )
  def prop(xp_hbm, src_hbm, dst_hbm, zeros_hbm, out_hbm,
           srcbuf, dstbuf, rows, shared, sem):
    c = lax.axis_index("c")
    s = lax.axis_index("s")
    wid = c * 16 + s
    r0 = pl.multiple_of(s * RPT, 8)
    pltpu.sync_copy(zeros_hbm.at[pl.ds(r0, RPT)], shared.at[pl.ds(r0, RPT)])
    pltpu.sync_copy(src_hbm.at[wid], srcbuf)
    pltpu.sync_copy(dst_hbm.at[wid], dstbuf)
    plsc.subcore_barrier()

    def body(j, carry):
      pltpu.async_copy(xp_hbm.at[srcbuf.at[j]], rows, sem).wait()
      pltpu.sync_copy(rows, shared.at[dstbuf.at[j]], add=True)
      return carry

    lax.fori_loop(0, NCH, body, 0)
    plsc.subcore_barrier()
    pltpu.sync_copy(shared.at[pl.ds(r0, RPT)],
                    out_hbm.at[c, pl.ds(r0, RPT)])

  return prop


_prop128 = _make_prop(D)
_prop16 = _make_prop(DL)


@functools.partial(
    pl.kernel,
    out_type=jax.ShapeDtypeStruct((2, NPAD, DL), jnp.float32),
    mesh=_sc_mesh,
    compiler_params=_sc_params,
    scratch_types=[
        pltpu.VMEM((NCH, CHUNK), jnp.int32),
        pltpu.VMEM((CHUNK, DL), jnp.float32),
        pltpu.VMEM_SHARED((NPAD, DL), jnp.float32),
        pltpu.SemaphoreType.DMA,
    ],
)
def _deg_kernel(dst_hbm, zeros_hbm, ones_hbm, out_hbm,
                dstbuf, onesv, shared, sem):
  c = lax.axis_index("c")
  s = lax.axis_index("s")
  wid = c * 16 + s
  r0 = pl.multiple_of(s * RPT, 8)
  pltpu.sync_copy(zeros_hbm.at[pl.ds(r0, RPT)], shared.at[pl.ds(r0, RPT)])
  pltpu.sync_copy(ones_hbm, onesv)
  pltpu.sync_copy(dst_hbm.at[wid], dstbuf)
  plsc.subcore_barrier()

  def body(j, carry):
    pltpu.sync_copy(onesv, shared.at[dstbuf.at[j]], add=True)
    return carry

  lax.fori_loop(0, NCH, body, 0)
  plsc.subcore_barrier()
  pltpu.sync_copy(shared.at[pl.ds(r0, RPT)],
                  out_hbm.at[c, pl.ds(r0, RPT)])


@functools.partial(
    pl.kernel,
    out_type=jax.ShapeDtypeStruct((NIDPAD, 2 * DL), jnp.float32),
    mesh=_sc_mesh,
    compiler_params=_sc_params,
    scratch_types=[
        pltpu.VMEM((IDS_PT,), jnp.int32),
        pltpu.VMEM((IDS_PT, 2 * DL), jnp.float32),
        pltpu.SemaphoreType.DMA,
    ],
)
def _gather_kernel(comb_hbm, ids_hbm, out_hbm, idbuf, rowsbuf, sem):
  c = lax.axis_index("c")
  s = lax.axis_index("s")
  wid = c * 16 + s
  pltpu.sync_copy(ids_hbm.at[wid], idbuf)
  pltpu.async_copy(comb_hbm.at[idbuf], rowsbuf, sem).wait()
  pltpu.sync_copy(rowsbuf, out_hbm.at[pl.ds(pl.multiple_of(wid * IDS_PT, 8),
                                            IDS_PT)])


BR = 640


def _mm1_body(x_ref, w_ref, degp_ref, xp_ref, dinv_ref):
  deg = degp_ref[0] + degp_ref[1] + 1.0
  dinv = lax.rsqrt(jnp.maximum(deg, 1.0))
  h = jnp.dot(x_ref[...], w_ref[...], preferred_element_type=jnp.float32)
  xp_ref[...] = h * dinv[:, :1]
  dinv_ref[...] = dinv


_mm1 = pl.pallas_call(
    _mm1_body,
    grid=(NPAD // BR,),
    in_specs=[
        pl.BlockSpec((BR, D), lambda i: (i, 0)),
        pl.BlockSpec((D, D), lambda i: (0, 0)),
        pl.BlockSpec((2, BR, DL), lambda i: (0, i, 0)),
    ],
    out_specs=[
        pl.BlockSpec((BR, D), lambda i: (i, 0)),
        pl.BlockSpec((BR, DL), lambda i: (i, 0)),
    ],
    out_shape=[
        jax.ShapeDtypeStruct((NPAD, D), jnp.float32),
        jax.ShapeDtypeStruct((NPAD, DL), jnp.float32),
    ],
)


def _mid_body(s1p_ref, xp1_ref, dinv_ref, b1_ref, w2_ref, xp2_ref):
  dinv = dinv_ref[...][:, :1]
  out1 = (s1p_ref[0] + s1p_ref[1] + xp1_ref[...]) * dinv + b1_ref[...]
  h1 = jnp.maximum(out1, 0.0)
  xp2_ref[...] = jnp.dot(h1, w2_ref[...],
                         preferred_element_type=jnp.float32) * dinv


_mid = pl.pallas_call(
    _mid_body,
    grid=(NPAD // BR,),
    in_specs=[
        pl.BlockSpec((2, BR, D), lambda i: (0, i, 0)),
        pl.BlockSpec((BR, D), lambda i: (i, 0)),
        pl.BlockSpec((BR, DL), lambda i: (i, 0)),
        pl.BlockSpec((1, D), lambda i: (0, 0)),
        pl.BlockSpec((D, DL), lambda i: (0, 0)),
    ],
    out_specs=pl.BlockSpec((BR, DL), lambda i: (i, 0)),
    out_shape=jax.ShapeDtypeStruct((NPAD, DL), jnp.float32),
)


def _pack_body(s2p_ref, xp2_ref, dinv_ref, b2_ref, lab_ref, comb_ref):
  dinv = dinv_ref[...][:, :1]
  out2 = (s2p_ref[0] + s2p_ref[1] + xp2_ref[...]) * dinv + b2_ref[...]
  labf = lax.bitcast_convert_type(lab_ref[...], jnp.float32)
  comb_ref[...] = jnp.concatenate([out2, labf], axis=1)


_pack = pl.pallas_call(
    _pack_body,
    grid=(NPAD // BR,),
    in_specs=[
        pl.BlockSpec((2, BR, DL), lambda i: (0, i, 0)),
        pl.BlockSpec((BR, DL), lambda i: (i, 0)),
        pl.BlockSpec((BR, DL), lambda i: (i, 0)),
        pl.BlockSpec((1, DL), lambda i: (0, 0)),
        pl.BlockSpec((BR, DL), lambda i: (i, 0)),
    ],
    out_specs=pl.BlockSpec((BR, 2 * DL), lambda i: (i, 0)),
    out_shape=jax.ShapeDtypeStruct((NPAD, 2 * DL), jnp.float32),
)


def _head_body(comb_ref, loss_ref):
  z = comb_ref[...]
  logits = z[:, :DL]
  lab = lax.bitcast_convert_type(z[:, DL:], jnp.int32)
  col = lax.broadcasted_iota(jnp.int32, (NIDPAD, DL), 1)
  picked = jnp.sum(jnp.where(col == lab, logits, 0.0), axis=1, keepdims=True)
  m = jnp.max(logits, axis=1, keepdims=True)
  lse = jnp.log(jnp.sum(jnp.exp(logits - m), axis=1, keepdims=True)) + m
  rowi = lax.broadcasted_iota(jnp.int32, (NIDPAD, 1), 0)
  mask = jnp.where(rowi < NID, 1.0, 0.0)
  loss_ref[...] = jnp.reshape(jnp.sum((lse - picked) * mask) / NID, (1, 1))


_head = pl.pallas_call(
    _head_body,
    grid=(1,),
    in_specs=[pl.BlockSpec((NIDPAD, 2 * DL), lambda i: (0, 0))],
    out_specs=pl.BlockSpec((1, 1), lambda i: (0, 0)),
    out_shape=jax.ShapeDtypeStruct((1, 1), jnp.float32),
)


def kernel(x, edge_index, node_ids, label_inds, W1, b1, W2, b2):
  x_pad = jnp.concatenate([x, jnp.zeros((NPAD - N, D), jnp.float32)], axis=0)
  pad_src = jnp.zeros((EPAD - E,), jnp.int32)
  pad_dst = (N + (jnp.arange(EPAD - E, dtype=jnp.int32) % (NPAD - N)))
  src = jnp.concatenate([edge_index[0], pad_src]).reshape(NW, NCH, CHUNK)
  dst = jnp.concatenate([edge_index[1], pad_dst]).reshape(NW, NCH, CHUNK)
  ids = jnp.concatenate(
      [node_ids.astype(jnp.int32),
       jnp.zeros((NIDPAD - NID,), jnp.int32)]).reshape(NW, IDS_PT)
  lab_pad = jnp.concatenate(
      [label_inds.astype(jnp.int32), jnp.zeros((NPAD - N,), jnp.int32)])
  lab16 = jnp.broadcast_to(lab_pad[:, None], (NPAD, DL))
  zeros128 = jnp.zeros((NPAD, D), jnp.float32)
  zeros16 = jnp.zeros((NPAD, DL), jnp.float32)
  ones16 = jnp.ones((CHUNK, DL), jnp.float32)

  degp = _deg_kernel(dst, zeros16, ones16)
  xp1, dinv16 = _mm1(x_pad, W1, degp)
  s1p = _prop128(xp1, src, dst, zeros128)
  xp2 = _mid(s1p, xp1, dinv16, b1.reshape(1, D), W2)
  s2p = _prop16(xp2, src, dst, zeros16)
  comb = _pack(s2p, xp2, dinv16, b2.reshape(1, DL), lab16)
  g = _gather_kernel(comb, ids)
  lossm = _head(g)
  return (lossm[0, 0], g[:NID, :DL])
